# Initial kernel scaffold; baseline (speedup 1.0000x reference)
import jax
import jax.numpy as jnp
from jax.experimental import pallas as pl


def kernel(z_e, codebook_weight):
    raise NotImplementedError("write your pallas kernel here")


# speedup vs baseline: 5.1708x; 5.1708x over previous
import functools

import jax
import jax.numpy as jnp
from jax.experimental import pallas as pl

_S = 10.0
_M = 0.1
_TOP_K = 8
_COS_M = float(jnp.cos(_M))
_SIN_M = float(jnp.sin(_M))

_K = 8192
_C = 256
_N = 8192
_RB = 256


def _arc_block(zt_ref, cb_ref, out_ref):
    z = zt_ref[...]
    inv_zn = 1.0 / jnp.maximum(
        jnp.sqrt(jnp.sum(z * z, axis=0, keepdims=True)), 1e-12)
    e = cb_ref[...]
    inv_en = 1.0 / jnp.maximum(
        jnp.sqrt(jnp.sum(e * e, axis=1, keepdims=True)), 1e-12)

    cos = jnp.dot(e, z, preferred_element_type=jnp.float32)
    cos = cos * inv_en * inv_zn

    rowsum = jnp.sum(jnp.exp(_S * cos), axis=1, keepdims=True)

    v = cos
    rem = jnp.full((_RB, 1), float(_TOP_K), dtype=jnp.float32)
    num = jnp.zeros((_RB, 1), dtype=jnp.float32)
    sub = jnp.zeros((_RB, 1), dtype=jnp.float32)
    for _ in range(_TOP_K):
        mx = jnp.max(v, axis=1, keepdims=True)
        eq = v == mx
        cnt = jnp.sum(eq.astype(jnp.float32), axis=1, keepdims=True)
        take = jnp.minimum(cnt, rem)
        rem = rem - take
        mc = jnp.clip(mx, -1.0 + 1e-07, 1.0 - 1e-07)
        mm = mc * _COS_M - jnp.sqrt(jnp.maximum(1.0 - mc * mc, 0.0)) * _SIN_M
        num = num + take * jnp.exp(_S * mm)
        sub = sub + take * jnp.exp(_S * mx)
        v = jnp.where(eq, -2.0, v)

    den = rowsum - sub + num
    blk = jnp.sum(jnp.log(num) - jnp.log(den)).reshape(1, 1)

    @pl.when(pl.program_id(0) == 0)
    def _init():
        out_ref[...] = jnp.zeros_like(out_ref)

    out_ref[...] += blk


@jax.jit
def kernel(z_e, codebook_weight):
    B, C, H, W = z_e.shape
    zt = jnp.transpose(z_e.reshape(B, C, H * W), (1, 0, 2)).reshape(C, B * H * W)

    total = pl.pallas_call(
        _arc_block,
        grid=(_K // _RB,),
        in_specs=[
            pl.BlockSpec((_C, _N), lambda i: (0, 0)),
            pl.BlockSpec((_RB, _C), lambda i: (i, 0)),
        ],
        out_specs=pl.BlockSpec((1, 1), lambda i: (0, 0)),
        out_shape=jax.ShapeDtypeStruct((1, 1), jnp.float32),
    )(zt, codebook_weight)

    return -(total[0, 0] / float(_K))


# speedup vs baseline: 7.8653x; 1.5211x over previous
import math

import jax
import jax.numpy as jnp
from jax.experimental import pallas as pl

_S = 10.0
_M = 0.1
_TOP_K = 8
_COS_M = math.cos(_M)
_SIN_M = math.sin(_M)

_K = 8192
_C = 256
_N = 8192
_RB = 256


_SB = 32
_NCH = _N // 128


def _strip_reduce(strip):
    acc = [jnp.full((_SB, 128), -2.0, dtype=jnp.float32)
           for _ in range(_TOP_K)]
    rs_acc = jnp.zeros((_SB, 128), dtype=jnp.float32)
    for c in range(_NCH):
        x = strip[:, c * 128:(c + 1) * 128]
        rs_acc = rs_acc + jnp.exp(_S * x)
        for i in range(_TOP_K):
            hi = jnp.maximum(acc[i], x)
            x = jnp.minimum(acc[i], x)
            acc[i] = hi
    cand = jnp.concatenate(acc, axis=1)
    rowsum = jnp.sum(rs_acc, axis=1, keepdims=True)

    rem = jnp.full((_SB, 1), float(_TOP_K), dtype=jnp.float32)
    num = jnp.zeros((_SB, 1), dtype=jnp.float32)
    sub = jnp.zeros((_SB, 1), dtype=jnp.float32)
    v = cand
    for _ in range(_TOP_K):
        mx = jnp.max(v, axis=1, keepdims=True)
        eq = v == mx
        cnt = jnp.sum(eq.astype(jnp.float32), axis=1, keepdims=True)
        take = jnp.minimum(cnt, rem)
        rem = rem - take
        mc = jnp.clip(mx, -1.0 + 1e-07, 1.0 - 1e-07)
        mm = mc * _COS_M - jnp.sqrt(jnp.maximum(1.0 - mc * mc, 0.0)) * _SIN_M
        num = num + take * jnp.exp(_S * mm)
        sub = sub + take * jnp.exp(_S * mx)
        v = jnp.where(eq, -2.0, v)
    return num, sub, rowsum


def _arc_block(zt_ref, cb_ref, out_ref):
    z = zt_ref[...]
    inv_zn = 1.0 / jnp.maximum(
        jnp.sqrt(jnp.sum(z * z, axis=0, keepdims=True)), 1e-12)
    e = cb_ref[...]
    inv_en = 1.0 / jnp.maximum(
        jnp.sqrt(jnp.sum(e * e, axis=1, keepdims=True)), 1e-12)

    cos = jnp.dot(e, z, preferred_element_type=jnp.float32)
    cos = cos * inv_en * inv_zn

    blk = jnp.zeros((1, 1), dtype=jnp.float32)
    for s in range(_RB // _SB):
        strip = cos[s * _SB:(s + 1) * _SB, :]
        num, sub, rowsum = _strip_reduce(strip)
        den = rowsum - sub + num
        blk = blk + jnp.sum(jnp.log(num) - jnp.log(den)).reshape(1, 1)

    @pl.when(pl.program_id(0) == 0)
    def _init():
        out_ref[...] = jnp.zeros_like(out_ref)

    out_ref[...] += blk


@jax.jit
def kernel(z_e, codebook_weight):
    B, C, H, W = z_e.shape
    zt = jnp.transpose(z_e.reshape(B, C, H * W), (1, 0, 2)).reshape(C, B * H * W)

    total = pl.pallas_call(
        _arc_block,
        grid=(_K // _RB,),
        in_specs=[
            pl.BlockSpec((_C, _N), lambda i: (0, 0)),
            pl.BlockSpec((_RB, _C), lambda i: (i, 0)),
        ],
        out_specs=pl.BlockSpec((1, 1), lambda i: (0, 0)),
        out_shape=jax.ShapeDtypeStruct((1, 1), jnp.float32),
    )(zt, codebook_weight)

    return -(total[0, 0] / float(_K))


# speedup vs baseline: 8.9985x; 1.1441x over previous
import math

import jax
import jax.numpy as jnp
from jax.experimental import pallas as pl
from jax.experimental.pallas import tpu as pltpu

_S = 10.0
_M = 0.1
_TOP_K = 8
_COS_M = math.cos(_M)
_SIN_M = math.sin(_M)

_K = 8192
_C = 256
_N = 8192
_RB = 256


_SB = 32
_NCH = _N // 128
_S_LOG2E = _S * math.log2(math.e)


def _strip_reduce(strip):
    acc = [jnp.full((_SB, 128), -2.0, dtype=jnp.float32)
           for _ in range(_TOP_K)]
    rs_acc = jnp.zeros((_SB, 128), dtype=jnp.float32)
    for c in range(_NCH):
        x = strip[:, c * 128:(c + 1) * 128]
        rs_acc = rs_acc + jnp.exp2(_S_LOG2E * x)
        for i in range(_TOP_K):
            hi = jnp.maximum(acc[i], x)
            x = jnp.minimum(acc[i], x)
            acc[i] = hi
    cand = jnp.concatenate(acc, axis=1)
    rowsum = jnp.sum(rs_acc, axis=1, keepdims=True)

    rem = jnp.full((_SB, 1), float(_TOP_K), dtype=jnp.float32)
    num = jnp.zeros((_SB, 1), dtype=jnp.float32)
    sub = jnp.zeros((_SB, 1), dtype=jnp.float32)
    v = cand
    for _ in range(_TOP_K):
        mx = jnp.max(v, axis=1, keepdims=True)
        eq = v == mx
        cnt = jnp.sum(eq.astype(jnp.float32), axis=1, keepdims=True)
        take = jnp.minimum(cnt, rem)
        rem = rem - take
        mc = jnp.clip(mx, -1.0 + 1e-07, 1.0 - 1e-07)
        mm = mc * _COS_M - jnp.sqrt(jnp.maximum(1.0 - mc * mc, 0.0)) * _SIN_M
        num = num + take * jnp.exp(_S * mm)
        sub = sub + take * jnp.exp(_S * mx)
        v = jnp.where(eq, -2.0, v)
    return num, sub, rowsum


def _arc_block(zt_ref, cb_ref, out_ref, zn_ref):
    @pl.when(pl.program_id(0) == 0)
    def _prep():
        z = zt_ref[...]
        inv_zn = jax.lax.rsqrt(
            jnp.maximum(jnp.sum(z * z, axis=0, keepdims=True), 1e-24))
        zn_ref[...] = z * inv_zn

    e = cb_ref[...]
    inv_en = jax.lax.rsqrt(
        jnp.maximum(jnp.sum(e * e, axis=1, keepdims=True), 1e-24))
    en = e * inv_en

    cos = jnp.dot(en, zn_ref[...], preferred_element_type=jnp.float32)

    blk = jnp.zeros((1, 1), dtype=jnp.float32)
    for s in range(_RB // _SB):
        strip = cos[s * _SB:(s + 1) * _SB, :]
        num, sub, rowsum = _strip_reduce(strip)
        den = rowsum - sub + num
        blk = blk + jnp.sum(jnp.log(num) - jnp.log(den)).reshape(1, 1)

    @pl.when(pl.program_id(0) == 0)
    def _init():
        out_ref[...] = jnp.zeros_like(out_ref)

    out_ref[...] += blk


@jax.jit
def kernel(z_e, codebook_weight):
    B, C, H, W = z_e.shape
    zt = jnp.transpose(z_e.reshape(B, C, H * W), (1, 0, 2)).reshape(C, B * H * W)

    total = pl.pallas_call(
        _arc_block,
        grid=(_K // _RB,),
        in_specs=[
            pl.BlockSpec((_C, _N), lambda i: (0, 0)),
            pl.BlockSpec((_RB, _C), lambda i: (i, 0)),
        ],
        out_specs=pl.BlockSpec((1, 1), lambda i: (0, 0)),
        out_shape=jax.ShapeDtypeStruct((1, 1), jnp.float32),
        scratch_shapes=[pltpu.VMEM((_C, _N), jnp.float32)],
    )(zt, codebook_weight)

    return -(total[0, 0] / float(_K))
